# baseline (speedup 1.0000x reference)
import functools

import jax
import jax.numpy as jnp
from jax import lax
from jax.experimental import pallas as pl
from jax.experimental.pallas import tpu as pltpu
from jax.experimental.pallas import tpu_sc as plsc

NC, NS, L = 2, 16, 16
NW = NC * NS

N = 200000
NP = 200704
B_, H_, W_ = 4, 256, 256
PIX = B_ * H_ * W_
SENT = PIX
PIX_PER_W = PIX // NW
CHUNK = 2048
GCH = 128
CIN, CHID, COUT = 128, 32, 128
TILE = 512


def _project_body(x_ref, y_ref, z_ref, bat_ref, cam_ref, pix_ref):
    x = x_ref[...]
    y = y_ref[...]
    z = z_ref[...]
    bat = bat_ref[...]
    u0 = jnp.zeros_like(x)
    v0 = jnp.zeros_like(x)
    w0 = jnp.zeros_like(x)
    for b in range(B_):
        c = [cam_ref[b * 12 + k] for k in range(12)]
        ub = x * c[0] + y * c[1] + z * c[2] + c[3]
        vb = x * c[4] + y * c[5] + z * c[6] + c[7]
        wb = x * c[8] + y * c[9] + z * c[10] + c[11]
        sel = bat == b
        u0 = jnp.where(sel, ub, u0)
        v0 = jnp.where(sel, vb, v0)
        w0 = jnp.where(sel, wb, w0)
    zz = jnp.abs(w0) + 1e-6
    u = jnp.tanh(u0 / zz)
    v = jnp.tanh(v0 / zz)
    up = jnp.clip(jnp.floor((u * 0.5 + 0.5) * W_), 0, W_ - 1).astype(jnp.int32)
    vp = jnp.clip(jnp.floor((v * 0.5 + 0.5) * H_), 0, H_ - 1).astype(jnp.int32)
    pix = bat * (H_ * W_) + vp * W_ + up
    pix_ref[...] = jnp.where(bat >= B_, SENT, pix)


def _project(x, y, z, bat, cam_flat):
    return pl.pallas_call(
        _project_body,
        out_shape=jax.ShapeDtypeStruct((NP // 128, 128), jnp.int32),
        in_specs=[
            pl.BlockSpec(memory_space=pltpu.VMEM),
            pl.BlockSpec(memory_space=pltpu.VMEM),
            pl.BlockSpec(memory_space=pltpu.VMEM),
            pl.BlockSpec(memory_space=pltpu.VMEM),
            pl.BlockSpec(memory_space=pltpu.SMEM),
        ],
        out_specs=pl.BlockSpec(memory_space=pltpu.VMEM),
    )(x, y, z, bat, cam_flat)


def _winner_body(pix_hbm, winner_hbm, idxc_hbm, buf_v, map_v, mapc_v):
    wid = lax.axis_index("s") * NC + lax.axis_index("c")
    base_pix = wid * PIX_PER_W
    lane = lax.iota(jnp.int32, L)
    shift_idx = jnp.minimum(lane + 1, L - 1)
    dnums = lax.GatherDimensionNumbers(
        offset_dims=(), collapsed_slice_dims=(0,), start_index_map=(0,))

    def init_body(i, _):
        map_v[pl.ds(i * L, L)] = jnp.full((L,), -1, jnp.int32)
        return 0
    lax.fori_loop(0, PIX_PER_W // L, init_body, 0)

    def chunk_body(ci, _):
        pltpu.sync_copy(pix_hbm.at[pl.ds(ci * CHUNK, CHUNK)], buf_v)

        def vreg_body(j, _):
            pix = buf_v[pl.ds(j * L, L)]
            key = pix * L + lane
            skey, _ = plsc.sort_key_val(key, key)
            pix_s = lax.shift_right_logical(skey, 4)
            lane_s = skey & (L - 1)
            n_s = ci * CHUNK + j * L + lane_s
            nxt = lax.gather(pix_s, shift_idx[:, None], dnums,
                             slice_sizes=(1,),
                             mode=lax.GatherScatterMode.PROMISE_IN_BOUNDS)
            run_last = (pix_s != nxt) | (lane == L - 1)
            inr = (pix_s >= base_pix) & (pix_s < base_pix + PIX_PER_W)
            plsc.store_scatter(map_v, [pix_s - base_pix], n_s,
                               mask=run_last & inr)
            return 0
        lax.fori_loop(0, CHUNK // L, vreg_body, 0)
        return 0
    lax.fori_loop(0, NP // CHUNK, chunk_body, 0)

    def clamp_body(i, _):
        v = map_v[pl.ds(i * L, L)]
        mapc_v[pl.ds(i * L, L)] = jnp.maximum(v, 0)
        return 0
    lax.fori_loop(0, PIX_PER_W // L, clamp_body, 0)

    pltpu.sync_copy(map_v, winner_hbm.at[pl.ds(base_pix, PIX_PER_W)])
    pltpu.sync_copy(mapc_v, idxc_hbm.at[pl.ds(base_pix, PIX_PER_W)])


@functools.cache
def _winner():
    return pl.kernel(
        _winner_body,
        out_type=(jax.ShapeDtypeStruct((PIX,), jnp.int32),
                  jax.ShapeDtypeStruct((PIX,), jnp.int32)),
        mesh=plsc.VectorSubcoreMesh(core_axis_name="c", subcore_axis_name="s"),
        scratch_types=[
            pltpu.VMEM((CHUNK,), jnp.int32),
            pltpu.VMEM((PIX_PER_W,), jnp.int32),
            pltpu.VMEM((PIX_PER_W,), jnp.int32),
        ],
        compiler_params=pltpu.CompilerParams(needs_layout_passes=False),
    )


def _gather_body(feat_hbm, idxc_hbm, g_hbm, idx_v, rows_v, sem):
    wid = lax.axis_index("s") * NC + lax.axis_index("c")
    base = wid * PIX_PER_W

    def chunk_body(ci, _):
        row0 = base + ci * GCH
        pltpu.sync_copy(idxc_hbm.at[pl.ds(row0, GCH)], idx_v)
        pltpu.async_copy(feat_hbm.at[idx_v], rows_v, sem).wait()
        pltpu.sync_copy(rows_v, g_hbm.at[pl.ds(row0, GCH)])
        return 0
    lax.fori_loop(0, PIX_PER_W // GCH, chunk_body, 0)


@functools.cache
def _gather():
    return pl.kernel(
        _gather_body,
        out_type=jax.ShapeDtypeStruct((PIX, CIN), jnp.float32),
        mesh=plsc.VectorSubcoreMesh(core_axis_name="c", subcore_axis_name="s"),
        scratch_types=[
            pltpu.VMEM((GCH,), jnp.int32),
            pltpu.VMEM((GCH, CIN), jnp.float32),
            pltpu.SemaphoreType.DMA,
        ],
    )


def _mlp_body(g_ref, win_ref, w1_ref, b1_ref, w2_ref, b2_ref, w3_ref, b3_ref,
              zet_ref, out_ref):
    x = g_ref[...]
    h = jnp.dot(x, w1_ref[...], preferred_element_type=jnp.float32)
    h = jnp.maximum(h + b1_ref[...], 0.0)
    h = jnp.dot(h, w2_ref[...], preferred_element_type=jnp.float32)
    h = jnp.maximum(h + b2_ref[...], 0.0)
    o = jnp.dot(h, w3_ref[...], preferred_element_type=jnp.float32)
    o = o + b3_ref[...]
    ot = o.T
    has = win_ref[0] >= 0
    out_ref[0] = jnp.where(has, ot, zet_ref[...])


def _mlp_fill(g, win3d, w1, b1, w2, b2, w3, b3, zet):
    nsteps = PIX // TILE
    return pl.pallas_call(
        _mlp_body,
        grid=(nsteps,),
        out_shape=jax.ShapeDtypeStruct((B_, COUT, H_ * W_), jnp.float32),
        in_specs=[
            pl.BlockSpec((TILE, CIN), lambda i: (i, 0)),
            pl.BlockSpec((1, 1, TILE), lambda i: (i, 0, 0)),
            pl.BlockSpec((CIN, CHID), lambda i: (0, 0)),
            pl.BlockSpec((1, CHID), lambda i: (0, 0)),
            pl.BlockSpec((CHID, COUT), lambda i: (0, 0)),
            pl.BlockSpec((1, COUT), lambda i: (0, 0)),
            pl.BlockSpec((COUT, COUT), lambda i: (0, 0)),
            pl.BlockSpec((1, COUT), lambda i: (0, 0)),
            pl.BlockSpec((COUT, 1), lambda i: (0, 0)),
        ],
        out_specs=pl.BlockSpec(
            (1, COUT, TILE), lambda i: (i // (H_ * W_ // TILE), 0,
                                        i % (H_ * W_ // TILE))),
    )(g, win3d, w1, b1, w2, b2, w3, b3, zet)


def kernel(pc_features, pc_pos, pc_batch, cam, W1, b1, W2, b2, W3, b3,
           zero_encoding, B, H, W):
    pad = NP - N
    x = jnp.pad(pc_pos[:, 0], (0, pad)).reshape(NP // 128, 128)
    y = jnp.pad(pc_pos[:, 1], (0, pad)).reshape(NP // 128, 128)
    z = jnp.pad(pc_pos[:, 2], (0, pad)).reshape(NP // 128, 128)
    bat = jnp.pad(jnp.clip(pc_batch, 0, B - 1).astype(jnp.int32), (0, pad),
                  constant_values=B_).reshape(NP // 128, 128)
    cam_flat = cam.reshape(-1)

    pix = _project(x, y, z, bat, cam_flat).reshape(NP)
    winner, idxc = _winner()(pix)
    g = _gather()(pc_features, idxc)
    win3d = winner.reshape(PIX // TILE, 1, TILE)
    out = _mlp_fill(g, win3d, W1, b1.reshape(1, CHID), W2,
                    b2.reshape(1, COUT), W3, b3.reshape(1, COUT),
                    zero_encoding.reshape(COUT, 1))
    return out.reshape(B_, COUT, H_, W_)


# speedup vs baseline: 1.0004x; 1.0004x over previous
import functools

import jax
import jax.numpy as jnp
from jax import lax
from jax.experimental import pallas as pl
from jax.experimental.pallas import tpu as pltpu
from jax.experimental.pallas import tpu_sc as plsc

NC, NS, L = 2, 16, 16
NW = NC * NS

N = 200000
NP = 200704
B_, H_, W_ = 4, 256, 256
PIX = B_ * H_ * W_
SENT = PIX
PIX_PER_W = PIX // NW
CHUNK = 2048
GCH = 128
CIN, CHID, COUT = 128, 32, 128
TILE = 512


def _project_body(x_ref, y_ref, z_ref, bat_ref, cam_ref, pix_ref):
    x = x_ref[...]
    y = y_ref[...]
    z = z_ref[...]
    bat = bat_ref[...]
    u0 = jnp.zeros_like(x)
    v0 = jnp.zeros_like(x)
    w0 = jnp.zeros_like(x)
    for b in range(B_):
        c = [cam_ref[b * 12 + k] for k in range(12)]
        ub = x * c[0] + y * c[1] + z * c[2] + c[3]
        vb = x * c[4] + y * c[5] + z * c[6] + c[7]
        wb = x * c[8] + y * c[9] + z * c[10] + c[11]
        sel = bat == b
        u0 = jnp.where(sel, ub, u0)
        v0 = jnp.where(sel, vb, v0)
        w0 = jnp.where(sel, wb, w0)
    zz = jnp.abs(w0) + 1e-6
    u = jnp.tanh(u0 / zz)
    v = jnp.tanh(v0 / zz)
    up = jnp.clip(jnp.floor((u * 0.5 + 0.5) * W_), 0, W_ - 1).astype(jnp.int32)
    vp = jnp.clip(jnp.floor((v * 0.5 + 0.5) * H_), 0, H_ - 1).astype(jnp.int32)
    pix = bat * (H_ * W_) + vp * W_ + up
    pix_ref[...] = jnp.where(bat >= B_, SENT, pix)


def _project(x, y, z, bat, cam_flat):
    return pl.pallas_call(
        _project_body,
        out_shape=jax.ShapeDtypeStruct((NP // 128, 128), jnp.int32),
        in_specs=[
            pl.BlockSpec(memory_space=pltpu.VMEM),
            pl.BlockSpec(memory_space=pltpu.VMEM),
            pl.BlockSpec(memory_space=pltpu.VMEM),
            pl.BlockSpec(memory_space=pltpu.VMEM),
            pl.BlockSpec(memory_space=pltpu.SMEM),
        ],
        out_specs=pl.BlockSpec(memory_space=pltpu.VMEM),
    )(x, y, z, bat, cam_flat)


def _winner_body(pix_hbm, winner_hbm, buf_v, map_v):
    wid = lax.axis_index("s") * NC + lax.axis_index("c")
    base_pix = wid * PIX_PER_W
    lane = lax.iota(jnp.int32, L)
    shift_idx = jnp.minimum(lane + 1, L - 1)
    dnums = lax.GatherDimensionNumbers(
        offset_dims=(), collapsed_slice_dims=(0,), start_index_map=(0,))

    def init_body(i, _):
        map_v[pl.ds(i * L, L)] = jnp.full((L,), -1, jnp.int32)
        return 0
    lax.fori_loop(0, PIX_PER_W // L, init_body, 0)

    def chunk_body(ci, _):
        pltpu.sync_copy(pix_hbm.at[pl.ds(ci * CHUNK, CHUNK)], buf_v)

        def vreg_body(j, _):
            pix = buf_v[pl.ds(j * L, L)]
            key = pix * L + lane
            skey, _ = plsc.sort_key_val(key, key)
            pix_s = lax.shift_right_logical(skey, 4)
            lane_s = skey & (L - 1)
            n_s = ci * CHUNK + j * L + lane_s
            nxt = lax.gather(pix_s, shift_idx[:, None], dnums,
                             slice_sizes=(1,),
                             mode=lax.GatherScatterMode.PROMISE_IN_BOUNDS)
            run_last = (pix_s != nxt) | (lane == L - 1)
            inr = (pix_s >= base_pix) & (pix_s < base_pix + PIX_PER_W)
            plsc.store_scatter(map_v, [pix_s - base_pix], n_s,
                               mask=run_last & inr)
            return 0
        lax.fori_loop(0, CHUNK // L, vreg_body, 0)
        return 0
    lax.fori_loop(0, NP // CHUNK, chunk_body, 0)

    pltpu.sync_copy(map_v, winner_hbm.at[pl.ds(base_pix, PIX_PER_W)])


@functools.cache
def _winner():
    return pl.kernel(
        _winner_body,
        out_type=jax.ShapeDtypeStruct((PIX,), jnp.int32),
        mesh=plsc.VectorSubcoreMesh(core_axis_name="c", subcore_axis_name="s"),
        scratch_types=[
            pltpu.VMEM((CHUNK,), jnp.int32),
            pltpu.VMEM((PIX_PER_W,), jnp.int32),
        ],
        compiler_params=pltpu.CompilerParams(needs_layout_passes=False),
    )


NCH = PIX_PER_W // GCH
RBUF = 4


def _gather_body(feat_hbm, win_hbm, g_hbm, idx_v, rows_v, sem_out, *sems):
    wid = lax.axis_index("s") * NC + lax.axis_index("c")
    base = wid * PIX_PER_W
    L_ = L

    pltpu.sync_copy(win_hbm.at[pl.ds(base, PIX_PER_W)], idx_v)

    def clamp_body(i, _):
        idx_v[pl.ds(i * L_, L_)] = jnp.maximum(idx_v[pl.ds(i * L_, L_)], 0)
        return 0
    lax.fori_loop(0, PIX_PER_W // L_, clamp_body, 0)

    def start_gather(g, b):
        pltpu.make_async_copy(
            feat_hbm.at[idx_v.at[pl.ds(g * GCH, GCH)]],
            rows_v.at[b], sems[b]).start()

    def finish_chunk(g, b):
        pltpu.make_async_copy(
            feat_hbm.at[idx_v.at[pl.ds(g * GCH, GCH)]],
            rows_v.at[b], sems[b]).wait()
        pltpu.async_copy(rows_v.at[b],
                         g_hbm.at[pl.ds(base + g * GCH, GCH)],
                         sem_out).wait()

    for b in range(RBUF):
        start_gather(b, b)

    def loop_body(ci, _):
        for b in range(RBUF):
            g = ci * RBUF + b
            finish_chunk(g, b)
            start_gather(g + RBUF, b)
        return 0
    lax.fori_loop(0, NCH // RBUF - 1, loop_body, 0)

    for b in range(RBUF):
        finish_chunk(NCH - RBUF + b, b)


@functools.cache
def _gather():
    return pl.kernel(
        _gather_body,
        out_type=jax.ShapeDtypeStruct((PIX, CIN), jnp.float32),
        mesh=plsc.VectorSubcoreMesh(core_axis_name="c", subcore_axis_name="s"),
        scratch_types=[
            pltpu.VMEM((PIX_PER_W,), jnp.int32),
            pltpu.VMEM((RBUF, GCH, CIN), jnp.float32),
            pltpu.SemaphoreType.DMA,
        ] + [pltpu.SemaphoreType.DMA] * RBUF,
    )


def _mlp_body(g_ref, win_ref, w1_ref, b1_ref, w2_ref, b2_ref, w3_ref, b3_ref,
              zet_ref, out_ref):
    x = g_ref[...]
    h = jnp.dot(x, w1_ref[...], preferred_element_type=jnp.float32)
    h = jnp.maximum(h + b1_ref[...], 0.0)
    h = jnp.dot(h, w2_ref[...], preferred_element_type=jnp.float32)
    h = jnp.maximum(h + b2_ref[...], 0.0)
    o = jnp.dot(h, w3_ref[...], preferred_element_type=jnp.float32)
    o = o + b3_ref[...]
    ot = o.T
    has = win_ref[0] >= 0
    out_ref[0] = jnp.where(has, ot, zet_ref[...])


def _mlp_fill(g, win3d, w1, b1, w2, b2, w3, b3, zet):
    nsteps = PIX // TILE
    return pl.pallas_call(
        _mlp_body,
        grid=(nsteps,),
        out_shape=jax.ShapeDtypeStruct((B_, COUT, H_ * W_), jnp.float32),
        in_specs=[
            pl.BlockSpec((TILE, CIN), lambda i: (i, 0)),
            pl.BlockSpec((1, 1, TILE), lambda i: (i, 0, 0)),
            pl.BlockSpec((CIN, CHID), lambda i: (0, 0)),
            pl.BlockSpec((1, CHID), lambda i: (0, 0)),
            pl.BlockSpec((CHID, COUT), lambda i: (0, 0)),
            pl.BlockSpec((1, COUT), lambda i: (0, 0)),
            pl.BlockSpec((COUT, COUT), lambda i: (0, 0)),
            pl.BlockSpec((1, COUT), lambda i: (0, 0)),
            pl.BlockSpec((COUT, 1), lambda i: (0, 0)),
        ],
        out_specs=pl.BlockSpec(
            (1, COUT, TILE), lambda i: (i // (H_ * W_ // TILE), 0,
                                        i % (H_ * W_ // TILE))),
    )(g, win3d, w1, b1, w2, b2, w3, b3, zet)


def kernel(pc_features, pc_pos, pc_batch, cam, W1, b1, W2, b2, W3, b3,
           zero_encoding, B, H, W):
    pad = NP - N
    x = jnp.pad(pc_pos[:, 0], (0, pad)).reshape(NP // 128, 128)
    y = jnp.pad(pc_pos[:, 1], (0, pad)).reshape(NP // 128, 128)
    z = jnp.pad(pc_pos[:, 2], (0, pad)).reshape(NP // 128, 128)
    bat = jnp.pad(jnp.clip(pc_batch, 0, B - 1).astype(jnp.int32), (0, pad),
                  constant_values=B_).reshape(NP // 128, 128)
    cam_flat = cam.reshape(-1)

    pix = _project(x, y, z, bat, cam_flat).reshape(NP)
    winner = _winner()(pix)
    g = _gather()(pc_features, winner)
    win3d = winner.reshape(PIX // TILE, 1, TILE)
    out = _mlp_fill(g, win3d, W1, b1.reshape(1, CHID), W2,
                    b2.reshape(1, COUT), W3, b3.reshape(1, COUT),
                    zero_encoding.reshape(COUT, 1))
    return out.reshape(B_, COUT, H_, W_)


# speedup vs baseline: 8.5647x; 8.5610x over previous
import functools

import jax
import jax.numpy as jnp
from jax import lax
from jax.experimental import pallas as pl
from jax.experimental.pallas import tpu as pltpu
from jax.experimental.pallas import tpu_sc as plsc

NC, NS, L = 2, 16, 16
NW = NC * NS

N = 200000
NP = 200704
B_, H_, W_ = 4, 256, 256
PIX = B_ * H_ * W_
SENT = PIX
PIX_PER_W = PIX // NW
CHUNK = 2048
GCH = 128
CIN, CHID, COUT = 128, 32, 128
TILE = 512


def _project_body(x_ref, y_ref, z_ref, bat_ref, cam_ref, pix_ref):
    x = x_ref[...]
    y = y_ref[...]
    z = z_ref[...]
    bat = bat_ref[...]
    u0 = jnp.zeros_like(x)
    v0 = jnp.zeros_like(x)
    w0 = jnp.zeros_like(x)
    for b in range(B_):
        c = [cam_ref[b * 12 + k] for k in range(12)]
        ub = x * c[0] + y * c[1] + z * c[2] + c[3]
        vb = x * c[4] + y * c[5] + z * c[6] + c[7]
        wb = x * c[8] + y * c[9] + z * c[10] + c[11]
        sel = bat == b
        u0 = jnp.where(sel, ub, u0)
        v0 = jnp.where(sel, vb, v0)
        w0 = jnp.where(sel, wb, w0)
    zz = jnp.abs(w0) + 1e-6
    u = jnp.tanh(u0 / zz)
    v = jnp.tanh(v0 / zz)
    up = jnp.clip(jnp.floor((u * 0.5 + 0.5) * W_), 0, W_ - 1).astype(jnp.int32)
    vp = jnp.clip(jnp.floor((v * 0.5 + 0.5) * H_), 0, H_ - 1).astype(jnp.int32)
    pix = bat * (H_ * W_) + vp * W_ + up
    pix_ref[...] = jnp.where(bat >= B_, SENT, pix)


def _project(x, y, z, bat, cam_flat):
    return pl.pallas_call(
        _project_body,
        out_shape=jax.ShapeDtypeStruct((NP // 128, 128), jnp.int32),
        in_specs=[
            pl.BlockSpec(memory_space=pltpu.VMEM),
            pl.BlockSpec(memory_space=pltpu.VMEM),
            pl.BlockSpec(memory_space=pltpu.VMEM),
            pl.BlockSpec(memory_space=pltpu.VMEM),
            pl.BlockSpec(memory_space=pltpu.SMEM),
        ],
        out_specs=pl.BlockSpec(memory_space=pltpu.VMEM),
    )(x, y, z, bat, cam_flat)


def _winner_body(pix_hbm, winner_hbm, buf_v, map_v):
    wid = lax.axis_index("s") * NC + lax.axis_index("c")
    base_pix = wid * PIX_PER_W
    lane = lax.iota(jnp.int32, L)
    shift_idx = jnp.minimum(lane + 1, L - 1)
    dnums = lax.GatherDimensionNumbers(
        offset_dims=(), collapsed_slice_dims=(0,), start_index_map=(0,))

    def init_body(i, _):
        map_v[pl.ds(i * L, L)] = jnp.full((L,), -1, jnp.int32)
        return 0
    lax.fori_loop(0, PIX_PER_W // L, init_body, 0)

    def chunk_body(ci, _):
        pltpu.sync_copy(pix_hbm.at[pl.ds(ci * CHUNK, CHUNK)], buf_v)

        def vreg_body(j, _):
            pix = buf_v[pl.ds(j * L, L)]
            key = pix * L + lane
            skey, _ = plsc.sort_key_val(key, key)
            pix_s = lax.shift_right_logical(skey, 4)
            lane_s = skey & (L - 1)
            n_s = ci * CHUNK + j * L + lane_s
            nxt = lax.gather(pix_s, shift_idx[:, None], dnums,
                             slice_sizes=(1,),
                             mode=lax.GatherScatterMode.PROMISE_IN_BOUNDS)
            run_last = (pix_s != nxt) | (lane == L - 1)
            inr = (pix_s >= base_pix) & (pix_s < base_pix + PIX_PER_W)
            plsc.store_scatter(map_v, [pix_s - base_pix], n_s,
                               mask=run_last & inr)
            return 0
        lax.fori_loop(0, CHUNK // L, vreg_body, 0)
        return 0
    lax.fori_loop(0, NP // CHUNK, chunk_body, 0)

    pltpu.sync_copy(map_v, winner_hbm.at[pl.ds(base_pix, PIX_PER_W)])


@functools.cache
def _winner():
    return pl.kernel(
        _winner_body,
        out_type=jax.ShapeDtypeStruct((PIX,), jnp.int32),
        mesh=plsc.VectorSubcoreMesh(core_axis_name="c", subcore_axis_name="s"),
        scratch_types=[
            pltpu.VMEM((CHUNK,), jnp.int32),
            pltpu.VMEM((PIX_PER_W,), jnp.int32),
        ],
        compiler_params=pltpu.CompilerParams(needs_layout_passes=False),
    )


NCH = PIX_PER_W // GCH
RBUF = 4


def _gather_body(feat_hbm, win_hbm, g_hbm, idx_v, rows_v, sem_out, *sems):
    wid = lax.axis_index("s") * NC + lax.axis_index("c")
    base = wid * PIX_PER_W
    L_ = L

    pltpu.sync_copy(win_hbm.at[pl.ds(base, PIX_PER_W)], idx_v)
    lane = lax.iota(jnp.int32, L_)

    def clamp_body(i, _):
        v = idx_v[pl.ds(i * L_, L_)]
        dummy = (base + i * L_ + lane) & 0x1FFFF
        idx_v[pl.ds(i * L_, L_)] = jnp.where(v < 0, dummy, v)
        return 0
    lax.fori_loop(0, PIX_PER_W // L_, clamp_body, 0)

    def start_gather(g, b):
        pltpu.make_async_copy(
            feat_hbm.at[idx_v.at[pl.ds(g * GCH, GCH)]],
            rows_v.at[b], sems[b]).start()

    def finish_chunk(g, b):
        pltpu.make_async_copy(
            feat_hbm.at[idx_v.at[pl.ds(g * GCH, GCH)]],
            rows_v.at[b], sems[b]).wait()
        pltpu.async_copy(rows_v.at[b],
                         g_hbm.at[pl.ds(base + g * GCH, GCH)],
                         sem_out).wait()

    for b in range(RBUF):
        start_gather(b, b)

    def loop_body(ci, _):
        for b in range(RBUF):
            g = ci * RBUF + b
            finish_chunk(g, b)
            start_gather(g + RBUF, b)
        return 0
    lax.fori_loop(0, NCH // RBUF - 1, loop_body, 0)

    for b in range(RBUF):
        finish_chunk(NCH - RBUF + b, b)


@functools.cache
def _gather():
    return pl.kernel(
        _gather_body,
        out_type=jax.ShapeDtypeStruct((PIX, CIN), jnp.float32),
        mesh=plsc.VectorSubcoreMesh(core_axis_name="c", subcore_axis_name="s"),
        scratch_types=[
            pltpu.VMEM((PIX_PER_W,), jnp.int32),
            pltpu.VMEM((RBUF, GCH, CIN), jnp.float32),
            pltpu.SemaphoreType.DMA,
        ] + [pltpu.SemaphoreType.DMA] * RBUF,
    )


def _mlp_body(g_ref, win_ref, w1_ref, b1_ref, w2_ref, b2_ref, w3_ref, b3_ref,
              zet_ref, out_ref):
    x = g_ref[...]
    h = jnp.dot(x, w1_ref[...], preferred_element_type=jnp.float32)
    h = jnp.maximum(h + b1_ref[...], 0.0)
    h = jnp.dot(h, w2_ref[...], preferred_element_type=jnp.float32)
    h = jnp.maximum(h + b2_ref[...], 0.0)
    o = jnp.dot(h, w3_ref[...], preferred_element_type=jnp.float32)
    o = o + b3_ref[...]
    ot = o.T
    has = win_ref[0] >= 0
    out_ref[0] = jnp.where(has, ot, zet_ref[...])


def _mlp_fill(g, win3d, w1, b1, w2, b2, w3, b3, zet):
    nsteps = PIX // TILE
    return pl.pallas_call(
        _mlp_body,
        grid=(nsteps,),
        out_shape=jax.ShapeDtypeStruct((B_, COUT, H_ * W_), jnp.float32),
        in_specs=[
            pl.BlockSpec((TILE, CIN), lambda i: (i, 0)),
            pl.BlockSpec((1, 1, TILE), lambda i: (i, 0, 0)),
            pl.BlockSpec((CIN, CHID), lambda i: (0, 0)),
            pl.BlockSpec((1, CHID), lambda i: (0, 0)),
            pl.BlockSpec((CHID, COUT), lambda i: (0, 0)),
            pl.BlockSpec((1, COUT), lambda i: (0, 0)),
            pl.BlockSpec((COUT, COUT), lambda i: (0, 0)),
            pl.BlockSpec((1, COUT), lambda i: (0, 0)),
            pl.BlockSpec((COUT, 1), lambda i: (0, 0)),
        ],
        out_specs=pl.BlockSpec(
            (1, COUT, TILE), lambda i: (i // (H_ * W_ // TILE), 0,
                                        i % (H_ * W_ // TILE))),
    )(g, win3d, w1, b1, w2, b2, w3, b3, zet)


def kernel(pc_features, pc_pos, pc_batch, cam, W1, b1, W2, b2, W3, b3,
           zero_encoding, B, H, W):
    pad = NP - N
    x = jnp.pad(pc_pos[:, 0], (0, pad)).reshape(NP // 128, 128)
    y = jnp.pad(pc_pos[:, 1], (0, pad)).reshape(NP // 128, 128)
    z = jnp.pad(pc_pos[:, 2], (0, pad)).reshape(NP // 128, 128)
    bat = jnp.pad(jnp.clip(pc_batch, 0, B - 1).astype(jnp.int32), (0, pad),
                  constant_values=B_).reshape(NP // 128, 128)
    cam_flat = cam.reshape(-1)

    pix = _project(x, y, z, bat, cam_flat).reshape(NP)
    winner = _winner()(pix)
    g = _gather()(pc_features, winner)
    win3d = winner.reshape(PIX // TILE, 1, TILE)
    out = _mlp_fill(g, win3d, W1, b1.reshape(1, CHID), W2,
                    b2.reshape(1, COUT), W3, b3.reshape(1, COUT),
                    zero_encoding.reshape(COUT, 1))
    return out.reshape(B_, COUT, H_, W_)
